# Initial kernel scaffold; baseline (speedup 1.0000x reference)
import jax
import jax.numpy as jnp
from jax.experimental import pallas as pl


def kernel(x, edge_index, W_proj, b_proj, W1_l, b1_l, W1_r, gamma, beta, W2_l, b2_l, W2_r):
    raise NotImplementedError("write your pallas kernel here")


# speedup vs baseline: 1.0388x; 1.0388x over previous
import functools

import jax
import jax.numpy as jnp
from jax import lax
from jax.experimental import pallas as pl
from jax.experimental.pallas import tpu as pltpu

N, E, D, H, C = 10000, 160000, 128, 6000, 1000
HP = 6144
CP = 1024
NCHUNK = CP // 128
EP = 163840
NPAD = 10240
BN_ROWS = 1000


def _t1_body(x_ref, wp_ref, bp_ref, xp_ref):
    acc = jnp.dot(x_ref[...], wp_ref[...], preferred_element_type=jnp.float32)
    xp_ref[...] = jnp.maximum(acc + bp_ref[0][None, :], 0.0)


def _t1(x, Wp, bp):
    return pl.pallas_call(
        _t1_body,
        grid=(N // BN_ROWS,),
        in_specs=[
            pl.BlockSpec((BN_ROWS, D), lambda i: (i, 0)),
            pl.BlockSpec((D, D), lambda i: (0, 0)),
            pl.BlockSpec((1, D), lambda i: (0, 0)),
        ],
        out_specs=pl.BlockSpec((BN_ROWS, D), lambda i: (i, 0)),
        out_shape=jax.ShapeDtypeStruct((N, D), jnp.float32),
    )(x, Wp, bp)


def _t2_body(sums_ref, cnt_ref, x_ref, w1l_ref, w1r_ref, b1_ref,
             h_ref, colsum_ref, colsq_ref):
    i = pl.program_id(1)
    s = sums_ref[0] + sums_ref[1]
    c = cnt_ref[0, :, 0] + cnt_ref[1, :, 0]
    inv = 1.0 / jnp.maximum(c, 1.0)
    agg = s * inv[:, None]
    hblk = (jnp.dot(agg, w1l_ref[...], preferred_element_type=jnp.float32)
            + jnp.dot(x_ref[...], w1r_ref[...], preferred_element_type=jnp.float32)
            + b1_ref[0][None, :])
    h_ref[...] = hblk

    @pl.when(i == 0)
    def _():
        colsum_ref[...] = jnp.zeros_like(colsum_ref)
        colsq_ref[...] = jnp.zeros_like(colsq_ref)

    ps = jnp.sum(hblk, axis=0)
    colsum_ref[...] += jnp.broadcast_to(ps[None, :], colsum_ref.shape)
    pq = jnp.sum(hblk * hblk, axis=0)
    colsq_ref[...] += jnp.broadcast_to(pq[None, :], colsq_ref.shape)


def _t2(sums, cnt2, x, W1l_p, W1r_p, b1_p):
    BH = 512
    grid = (HP // BH, N // BN_ROWS)
    return pl.pallas_call(
        _t2_body,
        grid=grid,
        in_specs=[
            pl.BlockSpec((2, BN_ROWS, D), lambda j, i: (0, i, 0)),
            pl.BlockSpec((2, BN_ROWS, 8), lambda j, i: (0, i, 0)),
            pl.BlockSpec((BN_ROWS, D), lambda j, i: (i, 0)),
            pl.BlockSpec((D, BH), lambda j, i: (0, j)),
            pl.BlockSpec((D, BH), lambda j, i: (0, j)),
            pl.BlockSpec((1, BH), lambda j, i: (0, j)),
        ],
        out_specs=[
            pl.BlockSpec((BN_ROWS, BH), lambda j, i: (i, j)),
            pl.BlockSpec((8, BH), lambda j, i: (0, j)),
            pl.BlockSpec((8, BH), lambda j, i: (0, j)),
        ],
        out_shape=[
            jax.ShapeDtypeStruct((N, HP), jnp.float32),
            jax.ShapeDtypeStruct((8, HP), jnp.float32),
            jax.ShapeDtypeStruct((8, HP), jnp.float32),
        ],
    )(sums, cnt2, x, W1l_p, W1r_p, b1_p)


def _t3_body(h_ref, colsum_ref, colsq_ref, gamma_ref, beta_ref,
             w2l_ref, w2r_ref, z_ref, hr_ref):
    k = pl.program_id(1)
    m = colsum_ref[0] * (1.0 / N)
    var = colsq_ref[0] * (1.0 / N) - m * m
    scale = gamma_ref[0] * lax.rsqrt(var + 1e-5)
    shift = beta_ref[0] - m * scale
    hn = jnp.maximum(h_ref[...] * scale[None, :] + shift[None, :], 0.0)

    @pl.when(k == 0)
    def _():
        z_ref[...] = jnp.zeros_like(z_ref)
        hr_ref[...] = jnp.zeros_like(hr_ref)

    z_ref[...] += jnp.dot(hn, w2l_ref[...], preferred_element_type=jnp.float32)
    hr_ref[...] += jnp.dot(hn, w2r_ref[...], preferred_element_type=jnp.float32)


def _t3(h, colsum, colsq, gamma_p, beta_p, W2l_p, W2r_p):
    BK = 1024
    grid = (N // BN_ROWS, HP // BK)
    return pl.pallas_call(
        _t3_body,
        grid=grid,
        in_specs=[
            pl.BlockSpec((BN_ROWS, BK), lambda i, k: (i, k)),
            pl.BlockSpec((8, BK), lambda i, k: (0, k)),
            pl.BlockSpec((8, BK), lambda i, k: (0, k)),
            pl.BlockSpec((1, BK), lambda i, k: (0, k)),
            pl.BlockSpec((1, BK), lambda i, k: (0, k)),
            pl.BlockSpec((BK, CP), lambda i, k: (k, 0)),
            pl.BlockSpec((BK, CP), lambda i, k: (k, 0)),
        ],
        out_specs=[
            pl.BlockSpec((BN_ROWS, CP), lambda i, k: (i, 0)),
            pl.BlockSpec((BN_ROWS, CP), lambda i, k: (i, 0)),
        ],
        out_shape=[
            jax.ShapeDtypeStruct((N, CP), jnp.float32),
            jax.ShapeDtypeStruct((N, CP), jnp.float32),
        ],
    )(h, colsum, colsq, gamma_p, beta_p, W2l_p, W2r_p)


def _t4_body(agg2_ref, cnt_ref, hr_ref, b2_ref, out_ref):
    c = cnt_ref[0, :, 0] + cnt_ref[1, :, 0]
    inv = 1.0 / jnp.maximum(c, 1.0)
    out_ref[...] = (agg2_ref[0] * inv[:, None] + b2_ref[0][None, :]
                    + hr_ref[...])


def _t4(agg2, cnt2, hr, b2_p):
    grid = (NCHUNK, N // BN_ROWS)
    return pl.pallas_call(
        _t4_body,
        grid=grid,
        in_specs=[
            pl.BlockSpec((1, BN_ROWS, 128), lambda j, i: (j, i, 0)),
            pl.BlockSpec((2, BN_ROWS, 8), lambda j, i: (0, i, 0)),
            pl.BlockSpec((BN_ROWS, 128), lambda j, i: (i, j)),
            pl.BlockSpec((1, 128), lambda j, i: (0, j)),
        ],
        out_specs=pl.BlockSpec((BN_ROWS, 128), lambda j, i: (i, j)),
        out_shape=jax.ShapeDtypeStruct((N, C), jnp.float32),
    )(agg2, cnt2, hr, b2_p)


def _agg_sums_xla(feat, src, dst):
    msg = jnp.take(feat, src, axis=0)
    s = jax.ops.segment_sum(msg, dst, num_segments=N)
    return s


def _counts_xla(dst):
    return jax.ops.segment_sum(jnp.ones((E,), jnp.float32), dst,
                               num_segments=N)


def kernel(x, edge_index, W_proj, b_proj, W1_l, b1_l, W1_r, gamma, beta,
           W2_l, b2_l, W2_r):
    src, dst = edge_index[0], edge_index[1]

    W1l_p = jnp.pad(W1_l, ((0, 0), (0, HP - H)))
    W1r_p = jnp.pad(W1_r, ((0, 0), (0, HP - H)))
    b1_p = jnp.pad(b1_l, (0, HP - H)).reshape(1, HP)
    gamma_p = jnp.pad(gamma, (0, HP - H)).reshape(1, HP)
    beta_p = jnp.pad(beta, (0, HP - H)).reshape(1, HP)
    W2l_p = jnp.pad(W2_l, ((0, HP - H), (0, CP - C)))
    W2r_p = jnp.pad(W2_r, ((0, HP - H), (0, CP - C)))
    b2_p = jnp.pad(b2_l, (0, CP - C)).reshape(1, CP)
    bp = b_proj.reshape(1, D)

    xp = _t1(x, W_proj, bp)

    s1 = _agg_sums_xla(xp, src, dst)
    cnt = _counts_xla(dst)
    sums1 = jnp.stack([s1, jnp.zeros_like(s1)])
    cnt8 = jnp.broadcast_to(cnt[:, None], (N, 8))
    cnt2 = jnp.stack([cnt8, jnp.zeros_like(cnt8)])

    h, colsum, colsq = _t2(sums1, cnt2, x, W1l_p, W1r_p, b1_p)
    z, hr = _t3(h, colsum, colsq, gamma_p, beta_p, W2l_p, W2r_p)

    s2 = _agg_sums_xla(z, src, dst)
    agg2 = s2.reshape(N, NCHUNK, 128).transpose(1, 0, 2)

    out = _t4(agg2, cnt2, hr, b2_p)
    return out


# speedup vs baseline: 2.5977x; 2.5005x over previous
import functools

import jax
import jax.numpy as jnp
from jax import lax
from jax.experimental import pallas as pl
from jax.experimental.pallas import tpu as pltpu

N, E, D, H, C = 10000, 160000, 128, 6000, 1000
HP = 6144
CP = 1024
NCHUNK = CP // 128
EP = 163840
NPAD = 10240
BN_ROWS = 1000


def _t1_body(x_ref, wp_ref, bp_ref, xp_ref):
    acc = jnp.dot(x_ref[...], wp_ref[...], preferred_element_type=jnp.float32)
    xp_ref[...] = jnp.maximum(acc + bp_ref[0][None, :], 0.0)


def _t1(x, Wp, bp):
    return pl.pallas_call(
        _t1_body,
        grid=(N // BN_ROWS,),
        in_specs=[
            pl.BlockSpec((BN_ROWS, D), lambda i: (i, 0)),
            pl.BlockSpec((D, D), lambda i: (0, 0)),
            pl.BlockSpec((1, D), lambda i: (0, 0)),
        ],
        out_specs=pl.BlockSpec((BN_ROWS, D), lambda i: (i, 0)),
        out_shape=jax.ShapeDtypeStruct((N, D), jnp.float32),
    )(x, Wp, bp)


def _t2_body(sums_ref, cnt_ref, x_ref, w1l_ref, w1r_ref, b1_ref,
             h_ref, colsum_ref, colsq_ref, invc_ref):
    i = pl.program_id(1)
    s = sums_ref[0] + sums_ref[1]
    c = cnt_ref[0, :, 0] + cnt_ref[1, :, 0]
    inv = 1.0 / jnp.maximum(c, 1.0)
    invc_ref[...] = jnp.broadcast_to(inv[:, None], invc_ref.shape)
    agg = s * inv[:, None]
    hblk = (jnp.dot(agg, w1l_ref[...], preferred_element_type=jnp.float32)
            + jnp.dot(x_ref[...], w1r_ref[...], preferred_element_type=jnp.float32)
            + b1_ref[0][None, :])
    h_ref[...] = hblk

    @pl.when(i == 0)
    def _():
        colsum_ref[...] = jnp.zeros_like(colsum_ref)
        colsq_ref[...] = jnp.zeros_like(colsq_ref)

    ps = jnp.sum(hblk, axis=0)
    colsum_ref[...] += jnp.broadcast_to(ps[None, :], colsum_ref.shape)
    pq = jnp.sum(hblk * hblk, axis=0)
    colsq_ref[...] += jnp.broadcast_to(pq[None, :], colsq_ref.shape)


def _t2(sums, cnt, x, W1l_p, W1r_p, b1_p):
    BH = 512
    grid = (HP // BH, N // BN_ROWS)
    return pl.pallas_call(
        _t2_body,
        grid=grid,
        in_specs=[
            pl.BlockSpec((2, BN_ROWS, D), lambda j, i: (0, i, 0)),
            pl.BlockSpec((2, BN_ROWS, D), lambda j, i: (0, i, 0)),
            pl.BlockSpec((BN_ROWS, D), lambda j, i: (i, 0)),
            pl.BlockSpec((D, BH), lambda j, i: (0, j)),
            pl.BlockSpec((D, BH), lambda j, i: (0, j)),
            pl.BlockSpec((1, BH), lambda j, i: (0, j)),
        ],
        out_specs=[
            pl.BlockSpec((BN_ROWS, BH), lambda j, i: (i, j)),
            pl.BlockSpec((8, BH), lambda j, i: (0, j)),
            pl.BlockSpec((8, BH), lambda j, i: (0, j)),
            pl.BlockSpec((BN_ROWS, 8), lambda j, i: (i, 0)),
        ],
        out_shape=[
            jax.ShapeDtypeStruct((N, HP), jnp.float32),
            jax.ShapeDtypeStruct((8, HP), jnp.float32),
            jax.ShapeDtypeStruct((8, HP), jnp.float32),
            jax.ShapeDtypeStruct((N, 8), jnp.float32),
        ],
    )(sums, cnt, x, W1l_p, W1r_p, b1_p)


def _t3_body(h_ref, colsum_ref, colsq_ref, gamma_ref, beta_ref,
             w2l_ref, w2r_ref, z_ref, hr_ref):
    k = pl.program_id(1)
    m = colsum_ref[0] * (1.0 / N)
    var = colsq_ref[0] * (1.0 / N) - m * m
    scale = gamma_ref[0] * lax.rsqrt(var + 1e-5)
    shift = beta_ref[0] - m * scale
    hn = jnp.maximum(h_ref[...] * scale[None, :] + shift[None, :], 0.0)

    @pl.when(k == 0)
    def _():
        z_ref[...] = jnp.zeros_like(z_ref)
        hr_ref[...] = jnp.zeros_like(hr_ref)

    z_ref[...] += jnp.dot(hn, w2l_ref[...], preferred_element_type=jnp.float32)
    hr_ref[...] += jnp.dot(hn, w2r_ref[...], preferred_element_type=jnp.float32)


def _t3(h, colsum, colsq, gamma_p, beta_p, W2l_p, W2r_p):
    BK = 1024
    grid = (N // BN_ROWS, HP // BK)
    return pl.pallas_call(
        _t3_body,
        grid=grid,
        in_specs=[
            pl.BlockSpec((BN_ROWS, BK), lambda i, k: (i, k)),
            pl.BlockSpec((8, BK), lambda i, k: (0, k)),
            pl.BlockSpec((8, BK), lambda i, k: (0, k)),
            pl.BlockSpec((1, BK), lambda i, k: (0, k)),
            pl.BlockSpec((1, BK), lambda i, k: (0, k)),
            pl.BlockSpec((BK, CP), lambda i, k: (k, 0)),
            pl.BlockSpec((BK, CP), lambda i, k: (k, 0)),
        ],
        out_specs=[
            pl.BlockSpec((BN_ROWS, CP), lambda i, k: (i, 0)),
            pl.BlockSpec((BN_ROWS, CP), lambda i, k: (i, 0)),
        ],
        out_shape=[
            jax.ShapeDtypeStruct((N, CP), jnp.float32),
            jax.ShapeDtypeStruct((N, CP), jnp.float32),
        ],
    )(h, colsum, colsq, gamma_p, beta_p, W2l_p, W2r_p)


def _t4_body(agg2_ref, invc_ref, hr_ref, b2_ref, out_ref):
    inv = invc_ref[:, 0]
    out_ref[...] = (agg2_ref[0] * inv[:, None] + b2_ref[0][None, :]
                    + hr_ref[...])


def _t4(agg2, invc, hr, b2_p):
    grid = (NCHUNK, N // BN_ROWS)
    return pl.pallas_call(
        _t4_body,
        grid=grid,
        in_specs=[
            pl.BlockSpec((1, BN_ROWS, 128), lambda j, i: (j, i, 0)),
            pl.BlockSpec((BN_ROWS, 8), lambda j, i: (i, 0)),
            pl.BlockSpec((BN_ROWS, 128), lambda j, i: (i, j)),
            pl.BlockSpec((1, 128), lambda j, i: (0, j)),
        ],
        out_specs=pl.BlockSpec((BN_ROWS, 128), lambda j, i: (i, j)),
        out_shape=jax.ShapeDtypeStruct((N, C), jnp.float32),
    )(agg2, invc, hr, b2_p)


from jax.experimental.pallas import tpu_sc as plsc

NCORE, NSUB = 2, 16
ROWS_A = (EP // 128) // (NCORE * NSUB)
ROWS_B = (EP // 128) // NSUB
SLICE = NPAD // NSUB
CC_PER_CORE = NCHUNK // NCORE


def _sc_mesh():
    return plsc.VectorSubcoreMesh(core_axis_name="c", subcore_axis_name="s")


def _zero_acc(zhbm, acc, base, nrows):
    for r in range(nrows // 128):
        pltpu.sync_copy(zhbm, acc.at[pl.ds(base + r * 128, 128)])


def _agg1_sc(xp, src2d, dst2d, zeros128, ones128):
    @functools.partial(
        pl.kernel,
        mesh=_sc_mesh(),
        out_type=[
            jax.ShapeDtypeStruct((NCORE, NPAD, 128), jnp.float32),
            jax.ShapeDtypeStruct((NCORE, NPAD, 128), jnp.float32),
        ],
        scratch_types=[
            pltpu.VMEM_SHARED((NPAD, 128), jnp.float32),
            pltpu.VMEM((ROWS_A, 128), jnp.int32),
            pltpu.VMEM((ROWS_A, 128), jnp.int32),
            pltpu.VMEM((128, 128), jnp.float32),
            pltpu.SemaphoreType.DMA,
        ],
    )
    def k(xp_hbm, src_hbm, dst_hbm, z128_hbm, o128_hbm,
          sums_hbm, cnt_hbm, acc, srcb, dstb, rows, sem):
        c = lax.axis_index("c")
        s = lax.axis_index("s")
        tid = c * NSUB + s
        pltpu.sync_copy(src_hbm.at[pl.ds(tid * ROWS_A, ROWS_A)], srcb)
        pltpu.sync_copy(dst_hbm.at[pl.ds(tid * ROWS_A, ROWS_A)], dstb)
        base = s * SLICE
        _zero_acc(z128_hbm, acc, base, SLICE)
        plsc.subcore_barrier()

        def body(g, carry):
            pltpu.async_copy(xp_hbm.at[srcb.at[g]], rows, sem).wait()
            pltpu.sync_copy(rows, acc.at[dstb.at[g]], add=True)
            return carry

        lax.fori_loop(0, ROWS_A, body, 0)
        plsc.subcore_barrier()
        pltpu.sync_copy(acc.at[pl.ds(base, SLICE)],
                        sums_hbm.at[c, pl.ds(base, SLICE)])
        plsc.subcore_barrier()

        _zero_acc(z128_hbm, acc, base, SLICE)
        pltpu.sync_copy(o128_hbm, rows)
        plsc.subcore_barrier()

        def body2(g, carry):
            pltpu.sync_copy(rows, acc.at[dstb.at[g]], add=True)
            return carry

        lax.fori_loop(0, ROWS_A, body2, 0)
        plsc.subcore_barrier()
        pltpu.sync_copy(acc.at[pl.ds(base, SLICE)],
                        cnt_hbm.at[c, pl.ds(base, SLICE)])

    return k(xp, src2d, dst2d, zeros128, ones128)


def _agg2_sc(zflat, src8, dst2d, zeros128):
    @functools.partial(
        pl.kernel,
        mesh=_sc_mesh(),
        out_type=jax.ShapeDtypeStruct((NCHUNK, NPAD, 128), jnp.float32),
        scratch_types=[
            pltpu.VMEM_SHARED((NPAD, 128), jnp.float32),
            pltpu.VMEM((ROWS_B, 128), jnp.int32),
            pltpu.VMEM((ROWS_B, 128), jnp.int32),
            pltpu.VMEM((128, 128), jnp.float32),
            pltpu.SemaphoreType.DMA,
        ],
    )
    def k(z_hbm, src_hbm, dst_hbm, z128_hbm,
          sums_hbm, acc, srcb, dstb, rows, sem):
        c = lax.axis_index("c")
        s = lax.axis_index("s")
        pltpu.sync_copy(dst_hbm.at[pl.ds(s * ROWS_B, ROWS_B)], dstb)
        base = s * SLICE
        for cc in range(CC_PER_CORE):
            chunk = c * CC_PER_CORE + cc
            pltpu.sync_copy(src_hbm.at[chunk, pl.ds(s * ROWS_B, ROWS_B)],
                            srcb)
            _zero_acc(z128_hbm, acc, base, SLICE)
            plsc.subcore_barrier()

            def body(g, carry):
                pltpu.async_copy(z_hbm.at[srcb.at[g]], rows, sem).wait()
                pltpu.sync_copy(rows, acc.at[dstb.at[g]], add=True)
                return carry

            lax.fori_loop(0, ROWS_B, body, 0)
            plsc.subcore_barrier()
            pltpu.sync_copy(acc.at[pl.ds(base, SLICE)],
                            sums_hbm.at[chunk, pl.ds(base, SLICE)])
            plsc.subcore_barrier()

    return k(zflat, src8, dst2d, zeros128)


def kernel(x, edge_index, W_proj, b_proj, W1_l, b1_l, W1_r, gamma, beta,
           W2_l, b2_l, W2_r):
    src, dst = edge_index[0], edge_index[1]

    W1l_p = jnp.pad(W1_l, ((0, 0), (0, HP - H)))
    W1r_p = jnp.pad(W1_r, ((0, 0), (0, HP - H)))
    b1_p = jnp.pad(b1_l, (0, HP - H)).reshape(1, HP)
    gamma_p = jnp.pad(gamma, (0, HP - H)).reshape(1, HP)
    beta_p = jnp.pad(beta, (0, HP - H)).reshape(1, HP)
    W2l_p = jnp.pad(W2_l, ((0, HP - H), (0, CP - C)))
    W2r_p = jnp.pad(W2_r, ((0, HP - H), (0, CP - C)))
    b2_p = jnp.pad(b2_l, (0, CP - C)).reshape(1, CP)
    bp = b_proj.reshape(1, D)

    pad = EP - E
    ar = jnp.arange(pad, dtype=jnp.int32)
    srcp = jnp.concatenate([src, (ar * 37) % N])
    dstp = jnp.concatenate([dst, N + (ar % (NPAD - N))])
    src2d = srcp.reshape(EP // 128, 128)
    dst2d = dstp.reshape(EP // 128, 128)
    src8 = (srcp[None, :] * NCHUNK
            + jnp.arange(NCHUNK, dtype=jnp.int32)[:, None]
            ).reshape(NCHUNK, EP // 128, 128)
    zeros128 = jnp.zeros((128, 128), jnp.float32)
    ones128 = jnp.ones((128, 128), jnp.float32)

    xp = _t1(x, W_proj, bp)

    sums1, cnt1 = _agg1_sc(xp, src2d, dst2d, zeros128, ones128)

    h, colsum, colsq, invc = _t2(sums1, cnt1, x, W1l_p, W1r_p, b1_p)
    z, hr = _t3(h, colsum, colsq, gamma_p, beta_p, W2l_p, W2r_p)

    agg2 = _agg2_sc(z.reshape(N * NCHUNK, 128), src8, dst2d, zeros128)

    out = _t4(agg2, invc, hr, b2_p)
    return out


# speedup vs baseline: 3.0843x; 1.1873x over previous
import functools

import jax
import jax.numpy as jnp
from jax import lax
from jax.experimental import pallas as pl
from jax.experimental.pallas import tpu as pltpu

N, E, D, H, C = 10000, 160000, 128, 6000, 1000
HP = 6144
CP = 1024
NCHUNK = CP // 128
EP = 163840
NPAD = 10240
BN_ROWS = 1000


def _t1_body(x_ref, wp_ref, bp_ref, xp_ref):
    acc = jnp.dot(x_ref[...], wp_ref[...], preferred_element_type=jnp.float32)
    xp_ref[...] = jnp.maximum(acc + bp_ref[0][None, :], 0.0)


def _t1(x, Wp, bp):
    return pl.pallas_call(
        _t1_body,
        grid=(N // BN_ROWS,),
        in_specs=[
            pl.BlockSpec((BN_ROWS, D), lambda i: (i, 0)),
            pl.BlockSpec((D, D), lambda i: (0, 0)),
            pl.BlockSpec((1, D), lambda i: (0, 0)),
        ],
        out_specs=pl.BlockSpec((BN_ROWS, D), lambda i: (i, 0)),
        out_shape=jax.ShapeDtypeStruct((N, D), jnp.float32),
    )(x, Wp, bp)


def _t2_body(sums_ref, cnt_ref, x_ref, w1l_ref, w1r_ref, b1_ref,
             h_ref, colsum_ref, colsq_ref, invc_ref):
    i = pl.program_id(1)
    s = sums_ref[0] + sums_ref[1]
    c = cnt_ref[0, :, 0] + cnt_ref[1, :, 0]
    inv = 1.0 / jnp.maximum(c, 1.0)
    invc_ref[...] = jnp.broadcast_to(inv[:, None], invc_ref.shape)
    agg = s * inv[:, None]
    hblk = (jnp.dot(agg, w1l_ref[...], preferred_element_type=jnp.float32)
            + jnp.dot(x_ref[...], w1r_ref[...], preferred_element_type=jnp.float32)
            + b1_ref[0][None, :])
    h_ref[...] = hblk

    @pl.when(i == 0)
    def _():
        colsum_ref[...] = jnp.zeros_like(colsum_ref)
        colsq_ref[...] = jnp.zeros_like(colsq_ref)

    ps = jnp.sum(hblk, axis=0)
    colsum_ref[...] += jnp.broadcast_to(ps[None, :], colsum_ref.shape)
    pq = jnp.sum(hblk * hblk, axis=0)
    colsq_ref[...] += jnp.broadcast_to(pq[None, :], colsq_ref.shape)


def _t2(sums, cnt, x, W1l_p, W1r_p, b1_p):
    BH = 512
    grid = (HP // BH, N // BN_ROWS)
    return pl.pallas_call(
        _t2_body,
        grid=grid,
        in_specs=[
            pl.BlockSpec((2, BN_ROWS, D), lambda j, i: (0, i, 0)),
            pl.BlockSpec((2, BN_ROWS, D), lambda j, i: (0, i, 0)),
            pl.BlockSpec((BN_ROWS, D), lambda j, i: (i, 0)),
            pl.BlockSpec((D, BH), lambda j, i: (0, j)),
            pl.BlockSpec((D, BH), lambda j, i: (0, j)),
            pl.BlockSpec((1, BH), lambda j, i: (0, j)),
        ],
        out_specs=[
            pl.BlockSpec((BN_ROWS, BH), lambda j, i: (i, j)),
            pl.BlockSpec((8, BH), lambda j, i: (0, j)),
            pl.BlockSpec((8, BH), lambda j, i: (0, j)),
            pl.BlockSpec((BN_ROWS, 8), lambda j, i: (i, 0)),
        ],
        out_shape=[
            jax.ShapeDtypeStruct((N, HP), jnp.float32),
            jax.ShapeDtypeStruct((8, HP), jnp.float32),
            jax.ShapeDtypeStruct((8, HP), jnp.float32),
            jax.ShapeDtypeStruct((N, 8), jnp.float32),
        ],
    )(sums, cnt, x, W1l_p, W1r_p, b1_p)


def _t3_body(h_ref, colsum_ref, colsq_ref, gamma_ref, beta_ref,
             w2l_ref, w2r_ref, z_ref, hr_ref):
    k = pl.program_id(1)
    m = colsum_ref[0] * (1.0 / N)
    var = colsq_ref[0] * (1.0 / N) - m * m
    scale = gamma_ref[0] * lax.rsqrt(var + 1e-5)
    shift = beta_ref[0] - m * scale
    hn = jnp.maximum(h_ref[...] * scale[None, :] + shift[None, :], 0.0)

    @pl.when(k == 0)
    def _():
        z_ref[...] = jnp.zeros_like(z_ref)
        hr_ref[...] = jnp.zeros_like(hr_ref)

    z_ref[...] += jnp.dot(hn, w2l_ref[...], preferred_element_type=jnp.float32)
    hr_ref[...] += jnp.dot(hn, w2r_ref[...], preferred_element_type=jnp.float32)


def _t3(h, colsum, colsq, gamma_p, beta_p, W2l_p, W2r_p):
    BK = 1024
    grid = (N // BN_ROWS, HP // BK)
    return pl.pallas_call(
        _t3_body,
        grid=grid,
        in_specs=[
            pl.BlockSpec((BN_ROWS, BK), lambda i, k: (i, k)),
            pl.BlockSpec((8, BK), lambda i, k: (0, k)),
            pl.BlockSpec((8, BK), lambda i, k: (0, k)),
            pl.BlockSpec((1, BK), lambda i, k: (0, k)),
            pl.BlockSpec((1, BK), lambda i, k: (0, k)),
            pl.BlockSpec((BK, CP), lambda i, k: (k, 0)),
            pl.BlockSpec((BK, CP), lambda i, k: (k, 0)),
        ],
        out_specs=[
            pl.BlockSpec((BN_ROWS, CP), lambda i, k: (i, 0)),
            pl.BlockSpec((BN_ROWS, CP), lambda i, k: (i, 0)),
        ],
        out_shape=[
            jax.ShapeDtypeStruct((N, CP), jnp.float32),
            jax.ShapeDtypeStruct((N, CP), jnp.float32),
        ],
    )(h, colsum, colsq, gamma_p, beta_p, W2l_p, W2r_p)


def _t4_body(agg2_ref, invc_ref, hr_ref, b2_ref, out_ref):
    inv = invc_ref[:, 0]
    out_ref[...] = (agg2_ref[0] * inv[:, None] + b2_ref[0][None, :]
                    + hr_ref[...])


def _t4(agg2, invc, hr, b2_p):
    grid = (NCHUNK, N // BN_ROWS)
    return pl.pallas_call(
        _t4_body,
        grid=grid,
        in_specs=[
            pl.BlockSpec((1, BN_ROWS, 128), lambda j, i: (j, i, 0)),
            pl.BlockSpec((BN_ROWS, 8), lambda j, i: (i, 0)),
            pl.BlockSpec((BN_ROWS, 128), lambda j, i: (i, j)),
            pl.BlockSpec((1, 128), lambda j, i: (0, j)),
        ],
        out_specs=pl.BlockSpec((BN_ROWS, 128), lambda j, i: (i, j)),
        out_shape=jax.ShapeDtypeStruct((N, C), jnp.float32),
    )(agg2, invc, hr, b2_p)


from jax.experimental.pallas import tpu_sc as plsc

NCORE, NSUB = 2, 16
ROWS_A = (EP // 128) // (NCORE * NSUB)
ROWS_B = (EP // 128) // NSUB
SLICE = NPAD // NSUB
CC_PER_CORE = NCHUNK // NCORE


def _sc_mesh():
    return plsc.VectorSubcoreMesh(core_axis_name="c", subcore_axis_name="s")


def _gather_scatter(tbl, srcb, dstb, acc, rows, sem0, sem1, nb):
    pltpu.async_copy(tbl.at[srcb.at[0]], rows.at[0], sem0)

    def pair(p, carry):
        g = 2 * p
        pltpu.async_copy(tbl.at[srcb.at[g + 1]], rows.at[1], sem1)
        pltpu.make_async_copy(tbl.at[srcb.at[g]], rows.at[0], sem0).wait()
        pltpu.sync_copy(rows.at[0], acc.at[dstb.at[g]], add=True)

        @pl.when(p + 1 < nb // 2)
        def _():
            pltpu.async_copy(tbl.at[srcb.at[g + 2]], rows.at[0], sem0)

        pltpu.make_async_copy(tbl.at[srcb.at[g + 1]], rows.at[1],
                              sem1).wait()
        pltpu.sync_copy(rows.at[1], acc.at[dstb.at[g + 1]], add=True)
        return carry

    lax.fori_loop(0, nb // 2, pair, 0)


def _zero_acc(zhbm, acc, base, nrows):
    for r in range(nrows // 128):
        pltpu.sync_copy(zhbm, acc.at[pl.ds(base + r * 128, 128)])


def _agg1_sc(xp, src2d, dst2d, zeros128, ones128):
    @functools.partial(
        pl.kernel,
        mesh=_sc_mesh(),
        out_type=[
            jax.ShapeDtypeStruct((NCORE, NPAD, 128), jnp.float32),
            jax.ShapeDtypeStruct((NCORE, NPAD, 128), jnp.float32),
        ],
        scratch_types=[
            pltpu.VMEM_SHARED((NPAD, 128), jnp.float32),
            pltpu.VMEM((ROWS_A, 128), jnp.int32),
            pltpu.VMEM((ROWS_A, 128), jnp.int32),
            pltpu.VMEM((2, 128, 128), jnp.float32),
            pltpu.SemaphoreType.DMA,
            pltpu.SemaphoreType.DMA,
        ],
    )
    def k(xp_hbm, src_hbm, dst_hbm, z128_hbm, o128_hbm,
          sums_hbm, cnt_hbm, acc, srcb, dstb, rows, sem0, sem1):
        c = lax.axis_index("c")
        s = lax.axis_index("s")
        tid = c * NSUB + s
        pltpu.sync_copy(src_hbm.at[pl.ds(tid * ROWS_A, ROWS_A)], srcb)
        pltpu.sync_copy(dst_hbm.at[pl.ds(tid * ROWS_A, ROWS_A)], dstb)
        base = s * SLICE
        _zero_acc(z128_hbm, acc, base, SLICE)
        plsc.subcore_barrier()

        _gather_scatter(xp_hbm, srcb, dstb, acc, rows, sem0, sem1, ROWS_A)
        plsc.subcore_barrier()
        pltpu.sync_copy(acc.at[pl.ds(base, SLICE)],
                        sums_hbm.at[c, pl.ds(base, SLICE)])
        plsc.subcore_barrier()

        _zero_acc(z128_hbm, acc, base, SLICE)
        pltpu.sync_copy(o128_hbm, rows.at[0])
        plsc.subcore_barrier()

        def fire(g, carry):
            pltpu.async_copy(rows.at[0], acc.at[dstb.at[g]], sem0,
                             add=True)
            return carry

        lax.fori_loop(0, ROWS_A, fire, 0)

        def drain(g, carry):
            pltpu.make_async_copy(rows.at[0], acc.at[dstb.at[g]],
                                  sem0).wait()
            return carry

        lax.fori_loop(0, ROWS_A, drain, 0)
        plsc.subcore_barrier()
        pltpu.sync_copy(acc.at[pl.ds(base, SLICE)],
                        cnt_hbm.at[c, pl.ds(base, SLICE)])

    return k(xp, src2d, dst2d, zeros128, ones128)


def _agg2_sc(zflat, src8, dst2d, zeros128):
    @functools.partial(
        pl.kernel,
        mesh=_sc_mesh(),
        out_type=jax.ShapeDtypeStruct((NCHUNK, NPAD, 128), jnp.float32),
        scratch_types=[
            pltpu.VMEM_SHARED((NPAD, 128), jnp.float32),
            pltpu.VMEM((ROWS_B // 2, 128), jnp.int32),
            pltpu.VMEM((ROWS_B // 2, 128), jnp.int32),
            pltpu.VMEM((2, 128, 128), jnp.float32),
            pltpu.SemaphoreType.DMA,
            pltpu.SemaphoreType.DMA,
        ],
    )
    def k(z_hbm, src_hbm, dst_hbm, z128_hbm,
          sums_hbm, acc, srcb, dstb, rows, sem0, sem1):
        c = lax.axis_index("c")
        s = lax.axis_index("s")
        base = s * SLICE
        for cc in range(CC_PER_CORE):
            chunk = c * CC_PER_CORE + cc
            _zero_acc(z128_hbm, acc, base, SLICE)
            plsc.subcore_barrier()
            for hf in range(2):
                rowbase = s * ROWS_B + hf * (ROWS_B // 2)
                pltpu.sync_copy(
                    src_hbm.at[chunk, pl.ds(rowbase, ROWS_B // 2)], srcb)
                pltpu.sync_copy(
                    dst_hbm.at[pl.ds(rowbase, ROWS_B // 2)], dstb)
                _gather_scatter(z_hbm, srcb, dstb, acc, rows,
                                sem0, sem1, ROWS_B // 2)
            plsc.subcore_barrier()
            pltpu.sync_copy(acc.at[pl.ds(base, SLICE)],
                            sums_hbm.at[chunk, pl.ds(base, SLICE)])
            plsc.subcore_barrier()

    return k(zflat, src8, dst2d, zeros128)


def kernel(x, edge_index, W_proj, b_proj, W1_l, b1_l, W1_r, gamma, beta,
           W2_l, b2_l, W2_r):
    src, dst = edge_index[0], edge_index[1]

    W1l_p = jnp.pad(W1_l, ((0, 0), (0, HP - H)))
    W1r_p = jnp.pad(W1_r, ((0, 0), (0, HP - H)))
    b1_p = jnp.pad(b1_l, (0, HP - H)).reshape(1, HP)
    gamma_p = jnp.pad(gamma, (0, HP - H)).reshape(1, HP)
    beta_p = jnp.pad(beta, (0, HP - H)).reshape(1, HP)
    W2l_p = jnp.pad(W2_l, ((0, HP - H), (0, CP - C)))
    W2r_p = jnp.pad(W2_r, ((0, HP - H), (0, CP - C)))
    b2_p = jnp.pad(b2_l, (0, CP - C)).reshape(1, CP)
    bp = b_proj.reshape(1, D)

    pad = EP - E
    ar = jnp.arange(pad, dtype=jnp.int32)
    srcp = jnp.concatenate([src, (ar * 37) % N])
    dstp = jnp.concatenate([dst, N + (ar % (NPAD - N))])
    src2d = srcp.reshape(EP // 128, 128)
    dst2d = dstp.reshape(EP // 128, 128)
    src8 = (srcp[None, :] * NCHUNK
            + jnp.arange(NCHUNK, dtype=jnp.int32)[:, None]
            ).reshape(NCHUNK, EP // 128, 128)
    zeros128 = jnp.zeros((128, 128), jnp.float32)
    ones128 = jnp.ones((128, 128), jnp.float32)

    xp = _t1(x, W_proj, bp)

    sums1, cnt1 = _agg1_sc(xp, src2d, dst2d, zeros128, ones128)

    h, colsum, colsq, invc = _t2(sums1, cnt1, x, W1l_p, W1r_p, b1_p)
    z, hr = _t3(h, colsum, colsq, gamma_p, beta_p, W2l_p, W2r_p)

    agg2 = _agg2_sc(z.reshape(N * NCHUNK, 128), src8, dst2d, zeros128)

    out = _t4(agg2, invc, hr, b2_p)
    return out
